# baseline (speedup 1.0000x reference)
import functools

import jax
import jax.numpy as jnp
from jax import lax
from jax.experimental import pallas as pl
from jax.experimental.pallas import tpu as pltpu
from jax.experimental.pallas import tpu_sc as plsc

_NC, _NS = 2, 16
_NW = _NC * _NS
_L = 128
_PL = 129
_J = 32


@functools.lru_cache(maxsize=None)
def _build_depad(vocab: int, dim: int):
    n_tiles = vocab // 8
    per_w = (n_tiles // (_NW * 8)) * 8
    n_extra = (n_tiles - per_w * _NW) // 8
    assert per_w * _NW + n_extra * 8 == n_tiles
    n_chunks = -(-(per_w + 8) // _J)

    mesh = plsc.VectorSubcoreMesh(core_axis_name="c", subcore_axis_name="s")

    @functools.partial(
        pl.kernel,
        out_type=jax.ShapeDtypeStruct((vocab * dim // _L, _L), jnp.float32),
        mesh=mesh,
        scratch_types=[
            pltpu.VMEM((2, _J, 8, dim), jnp.float32),
            pltpu.VMEM((2, 2 * _J, _L), jnp.float32),
            pltpu.SemaphoreType.DMA((2,)),
            pltpu.SemaphoreType.DMA((2,)),
        ],
        compiler_params=pltpu.CompilerParams(
            use_tc_tiling_on_sc=True, needs_layout_passes=False),
    )
    def depad_kernel(wp, tbl, sins, souts, isems, osems):
        wid = lax.axis_index("s") * _NC + lax.axis_index("c")
        start = wid * per_w + 8 * jnp.minimum(wid, n_extra)
        cnt = per_w + jnp.where(wid < n_extra, 8, 0)

        def j0_of(c):
            return start + jnp.minimum(c * _J, cnt - _J)

        def fire_in(c, b):
            pltpu.async_copy(wp.at[pl.ds(j0_of(c), _J)], sins.at[b],
                             isems.at[b])

        def wait_in(b):
            pltpu.make_async_copy(wp.at[pl.ds(0, _J)], sins.at[b],
                                  isems.at[b]).wait()

        def bridge(b):
            for r in range(2 * _J):
                for h in range(8):
                    x = sins[b, r // 2, (r % 2) * 4 + h // 2,
                             pl.ds((h % 2) * 16, 16)]
                    souts[b, r, pl.ds(16 * h, 16)] = x

        def fire_out(c, b):
            pltpu.async_copy(souts.at[b], tbl.at[pl.ds(j0_of(c) * 2, 2 * _J)],
                             osems.at[b])

        def wait_out(b):
            pltpu.make_async_copy(souts.at[b], tbl.at[pl.ds(0, 2 * _J)],
                                  osems.at[b]).wait()

        fire_in(jnp.int32(0), 0)
        fire_in(jnp.int32(1), 1)

        def cbody(c, carry):
            b = c & 1
            wait_in(b)

            @pl.when(c >= 2)
            def _():
                wait_out(b)

            bridge(b)
            fire_out(c, b)
            fire_in(jnp.minimum(c + 2, n_chunks - 1), b)
            return carry

        lax.fori_loop(0, n_chunks, cbody, 0, unroll=False)

        for b in range(2):
            wait_in(b)
            wait_out(b)

    return depad_kernel


@functools.lru_cache(maxsize=None)
def _build_gather(batch: int, hist: int, vocab: int, dim: int):
    assert batch == _NW * _L and hist % 8 == 0 and dim == 32
    n_tr = hist // 8
    n_bc = batch // _L

    mesh = plsc.VectorSubcoreMesh(core_axis_name="c", subcore_axis_name="s")

    @functools.partial(
        pl.kernel,
        out_type=jax.ShapeDtypeStruct((hist, 4, n_bc, 8, _L), jnp.float32),
        mesh=mesh,
        scratch_types=[
            pltpu.VMEM((n_tr, 8, _L), jnp.int32),
            pltpu.VMEM((2, _L, dim), jnp.float32),
            pltpu.VMEM((2, 4, 9, _PL), jnp.float32),
            pltpu.SemaphoreType.DMA((2,)),
            pltpu.SemaphoreType.DMA((2,)),
        ],
        compiler_params=pltpu.CompilerParams(
            use_tc_tiling_on_sc=False, needs_layout_passes=False),
    )
    def gather_kernel(idx4, tbl, out5, idx_v, gbufs, obs, gsems, wsems):
        wid = lax.axis_index("s") * _NC + lax.axis_index("c")
        pltpu.sync_copy(idx4.at[:, wid], idx_v)

        iot = lax.iota(jnp.int32, 16)
        dr0 = lax.shift_right_logical(iot, 3)
        s0 = iot & 7
        dr1 = lax.shift_right_logical(iot + 16, 3)
        s1 = (iot + 16) & 7

        def fire_gather(t, b):
            tr = lax.shift_right_logical(t, 3)
            s = t & 7
            pltpu.async_copy(tbl.at[idx_v.at[tr, s]], gbufs.at[b], gsems.at[b])

        def wait_gather(b):
            pltpu.make_async_copy(
                tbl.at[idx_v.at[0, 0]], gbufs.at[b], gsems.at[b]).wait()

        def select(b):
            bb = jnp.full((16,), b, jnp.int32)
            for l in range(_L):
                lv = jnp.full((16,), l, jnp.int32)
                x0 = gbufs[b, l, pl.ds(0, 16)]
                x1 = gbufs[b, l, pl.ds(16, 16)]
                plsc.store_scatter(obs, [bb, dr0, s0, lv], x0)
                plsc.store_scatter(obs, [bb, dr1, s1, lv], x1)

        def fire_wb(t, b):
            pltpu.async_copy(
                obs.at[b, :, pl.ds(0, 8), pl.ds(0, _L)],
                out5.at[t, pl.ds(0, 4), wid], wsems.at[b])

        def wait_wb(b):
            pltpu.make_async_copy(
                obs.at[b, :, pl.ds(0, 8), pl.ds(0, _L)],
                out5.at[0, pl.ds(0, 4), 0], wsems.at[b]).wait()

        fire_gather(jnp.int32(0), 0)
        fire_gather(jnp.int32(1), 1)

        def tbody(t, carry):
            b = t & 1
            wait_gather(b)

            @pl.when(t >= 2)
            def _():
                wait_wb(b)

            select(b)
            fire_wb(t, b)
            fire_gather(jnp.minimum(t + 2, hist - 1), b)
            return carry

        lax.fori_loop(0, hist, tbody, 0, unroll=False)

        for b in range(2):
            wait_gather(b)
            wait_wb(b)

    return gather_kernel


def kernel(inputs, weight):
    batch, hist = inputs.shape
    vocab, dim = weight.shape
    idx4 = (inputs.astype(jnp.int32)
            .reshape(batch // _L, _L, hist // 8, 8)
            .transpose(2, 0, 3, 1))
    wp = weight.reshape(vocab // 8, 8, dim)
    tbl2 = _build_depad(vocab, dim)(wp)
    tblD = tbl2.reshape(vocab, dim)
    out5 = _build_gather(batch, hist, vocab, dim)(idx4, tblD)
    y3 = out5.transpose(0, 1, 3, 2, 4).reshape(hist, dim, batch)
    return y3.transpose(2, 0, 1)


# speedup vs baseline: 1.0570x; 1.0570x over previous
import functools

import jax
import jax.numpy as jnp
from jax import lax
from jax.experimental import pallas as pl
from jax.experimental.pallas import tpu as pltpu
from jax.experimental.pallas import tpu_sc as plsc

_NC, _NS = 2, 16
_NW = _NC * _NS
_L = 128
_PL = 129
_J = 32


@functools.lru_cache(maxsize=None)
def _build_depad(vocab: int, dim: int):
    n_tiles = vocab // 8
    per_w = (n_tiles // (_NW * 8)) * 8
    n_extra = (n_tiles - per_w * _NW) // 8
    assert per_w * _NW + n_extra * 8 == n_tiles
    n_chunks = -(-(per_w + 8) // _J)

    mesh = plsc.VectorSubcoreMesh(core_axis_name="c", subcore_axis_name="s")

    @functools.partial(
        pl.kernel,
        out_type=jax.ShapeDtypeStruct((vocab * dim // _L, _L), jnp.float32),
        mesh=mesh,
        scratch_types=[
            pltpu.VMEM((2, _J, 8, dim), jnp.float32),
            pltpu.VMEM((2, 2 * _J, _L), jnp.float32),
            pltpu.SemaphoreType.DMA((2,)),
            pltpu.SemaphoreType.DMA((2,)),
        ],
        compiler_params=pltpu.CompilerParams(
            use_tc_tiling_on_sc=True, needs_layout_passes=False),
    )
    def depad_kernel(wp, tbl, sins, souts, isems, osems):
        wid = lax.axis_index("s") * _NC + lax.axis_index("c")
        start = wid * per_w + 8 * jnp.minimum(wid, n_extra)
        cnt = per_w + jnp.where(wid < n_extra, 8, 0)

        def j0_of(c):
            return start + jnp.minimum(c * _J, cnt - _J)

        def fire_in(c, b):
            pltpu.async_copy(wp.at[pl.ds(j0_of(c), _J)], sins.at[b],
                             isems.at[b])

        def wait_in(b):
            pltpu.make_async_copy(wp.at[pl.ds(0, _J)], sins.at[b],
                                  isems.at[b]).wait()

        def bridge(b):
            for r in range(2 * _J):
                for h in range(8):
                    x = sins[b, r // 2, (r % 2) * 4 + h // 2,
                             pl.ds((h % 2) * 16, 16)]
                    souts[b, r, pl.ds(16 * h, 16)] = x

        def fire_out(c, b):
            pltpu.async_copy(souts.at[b], tbl.at[pl.ds(j0_of(c) * 2, 2 * _J)],
                             osems.at[b])

        def wait_out(b):
            pltpu.make_async_copy(souts.at[b], tbl.at[pl.ds(0, 2 * _J)],
                                  osems.at[b]).wait()

        fire_in(jnp.int32(0), 0)
        fire_in(jnp.int32(1), 1)

        def cbody(c, carry):
            b = c & 1
            wait_in(b)

            @pl.when(c >= 2)
            def _():
                wait_out(b)

            bridge(b)
            fire_out(c, b)
            fire_in(jnp.minimum(c + 2, n_chunks - 1), b)
            return carry

        lax.fori_loop(0, n_chunks, cbody, 0, unroll=False)

        for b in range(2):
            wait_in(b)
            wait_out(b)

    return depad_kernel


@functools.lru_cache(maxsize=None)
def _build_gather(batch: int, hist: int, vocab: int, dim: int):
    assert batch == _NW * _L and hist % 8 == 0 and dim == 32
    n_tr = hist // 8
    n_bc = batch // _L

    mesh = plsc.VectorSubcoreMesh(core_axis_name="c", subcore_axis_name="s")

    @functools.partial(
        pl.kernel,
        out_type=jax.ShapeDtypeStruct((hist, 4, n_bc, 8, _L), jnp.float32),
        mesh=mesh,
        scratch_types=[
            pltpu.VMEM((n_tr, 8, _L), jnp.int32),
            pltpu.VMEM((2, _L, dim), jnp.float32),
            pltpu.VMEM((2, 4, 8, _PL), jnp.float32),
            pltpu.SemaphoreType.DMA((2,)),
            pltpu.SemaphoreType.DMA((2,)),
        ],
        compiler_params=pltpu.CompilerParams(
            use_tc_tiling_on_sc=False, needs_layout_passes=False),
    )
    def gather_kernel(idx4, tbl, out5, idx_v, gbufs, obs, gsems, wsems):
        wid = lax.axis_index("s") * _NC + lax.axis_index("c")
        pltpu.sync_copy(idx4.at[:, wid], idx_v)

        iot = lax.iota(jnp.int32, 16)
        dr0 = lax.shift_right_logical(iot, 3)
        s0 = iot & 7
        dr1 = lax.shift_right_logical(iot + 16, 3)
        s1 = (iot + 16) & 7

        def fire_gather(t, b):
            tr = lax.shift_right_logical(t, 3)
            s = t & 7
            pltpu.async_copy(tbl.at[idx_v.at[tr, s]], gbufs.at[b], gsems.at[b])

        def wait_gather(b):
            pltpu.make_async_copy(
                tbl.at[idx_v.at[0, 0]], gbufs.at[b], gsems.at[b]).wait()

        def select(b):
            bb = jnp.full((16,), b, jnp.int32)
            for l in range(_L):
                lv = jnp.full((16,), l, jnp.int32)
                x0 = gbufs[b, l, pl.ds(0, 16)]
                x1 = gbufs[b, l, pl.ds(16, 16)]
                plsc.store_scatter(obs, [bb, dr0, s0, lv], x0)
                plsc.store_scatter(obs, [bb, dr1, s1, lv], x1)

        def fire_wb(t, b):
            pltpu.async_copy(
                obs.at[b, :, :, pl.ds(0, _L)],
                out5.at[t, pl.ds(0, 4), wid], wsems.at[b])

        def wait_wb(b):
            pltpu.make_async_copy(
                obs.at[b, :, :, pl.ds(0, _L)],
                out5.at[0, pl.ds(0, 4), 0], wsems.at[b]).wait()

        fire_gather(jnp.int32(0), 0)
        fire_gather(jnp.int32(1), 1)

        def tbody(t, carry):
            b = t & 1
            wait_gather(b)

            @pl.when(t >= 2)
            def _():
                wait_wb(b)

            select(b)
            fire_wb(t, b)
            fire_gather(jnp.minimum(t + 2, hist - 1), b)
            return carry

        lax.fori_loop(0, hist, tbody, 0, unroll=False)

        for b in range(2):
            wait_gather(b)
            wait_wb(b)

    return gather_kernel


def kernel(inputs, weight):
    batch, hist = inputs.shape
    vocab, dim = weight.shape
    idx4 = (inputs.astype(jnp.int32)
            .reshape(batch // _L, _L, hist // 8, 8)
            .transpose(2, 0, 3, 1))
    wp = weight.reshape(vocab // 8, 8, dim)
    tbl2 = _build_depad(vocab, dim)(wp)
    tblD = tbl2.reshape(vocab, dim)
    out5 = _build_gather(batch, hist, vocab, dim)(idx4, tblD)
    y3 = out5.transpose(0, 1, 3, 2, 4).reshape(hist, dim, batch)
    return y3.transpose(2, 0, 1)
